# baseline (speedup 1.0000x reference)
import functools

import jax
import jax.numpy as jnp
from jax import lax
from jax.experimental import pallas as pl
from jax.experimental.pallas import tpu as pltpu
from jax.experimental.pallas import tpu_sc as plsc

_NC = 2
_NS = 16
_L = 16


def _sigmoid(z):
    d = 1.0 + jnp.exp(jnp.minimum(-z, 87.0))
    rc = lax.bitcast_convert_type(
        jnp.int32(0x7EF311C3) - lax.bitcast_convert_type(d, jnp.int32),
        jnp.float32)
    for _ in range(3):
        rc = rc * (2.0 - d * rc)
    return rc


def _sc_body(nrows, ngroups, x_hbm, site_hbm, app_hbm, w_hbm, out_hbm,
             xv, sv, av, wv, lut, ov, ssp, asp, wsp, sem, sem2):
    wid = lax.axis_index("s") * _NC + lax.axis_index("c")
    xcp = pltpu.async_copy(
        x_hbm.at[pl.ds(wid * (nrows * 15), nrows * 15)], xv, sem)

    @pl.when(lax.axis_index("s") == 0)
    def _():
        c1 = pltpu.async_copy(site_hbm, ssp, sem2)
        c2 = pltpu.async_copy(app_hbm, asp, sem2)
        c3 = pltpu.async_copy(w_hbm, wsp, sem2)
        c1.wait()
        c2.wait()
        c3.wait()
    plsc.subcore_barrier()
    l1 = pltpu.async_copy(ssp, sv, sem2)
    l2 = pltpu.async_copy(asp, av, sem2)
    l3 = pltpu.async_copy(wsp, wv, sem2)
    l1.wait()
    l2.wait()
    l3.wait()

    def splat(val):
        return jnp.broadcast_to(val, (_L,))

    wvec = [wv[pl.ds(16 * q, _L)] for q in range(7)]

    def wsc(flat):
        return wvec[flat // _L][flat % _L]

    w6 = [wsc(78 + j) for j in range(6)]

    vsp = []
    for d in range(12):
        acc = w6[0] * wsc(d)
        for j in range(1, 6):
            acc = acc + w6[j] * wsc(j * 12 + d)
        vsp.append(splat(acc))

    lane = jax.lax.iota(jnp.int32, _L)
    for half in range(2):
        row6 = (lane + half * _L) * 6
        ss = jnp.zeros((_L,), jnp.float32)
        aa = jnp.zeros((_L,), jnp.float32)
        for d in range(6):
            if half == 0:
                ss = ss + vsp[d] * plsc.load_gather(sv, [row6 + d])
            else:
                ss = ss + vsp[d] * plsc.load_gather(
                    sv, [jnp.minimum(row6 + d, 143)])
            aa = aa + vsp[6 + d] * plsc.load_gather(av, [row6 + d])
        lut[pl.ds(half * _L, _L)] = ss
        lut[pl.ds(32 + half * _L, _L)] = aa

    plsc.subcore_barrier()

    cs = wsc(97)
    for i in range(6):
        cs = cs + w6[i] * wsc(72 + i)
    c16 = splat(cs)
    wd = [splat(wsc(84 + k)) for k in range(13)]

    lane15 = lane * 15
    xcp.wait()

    _UNROLL = 4

    def group(gq, carry):
        for u in range(_UNROLL):
            g = gq * _UNROLL + u
            fid = lane15 + g * (_L * 15)
            si = plsc.load_gather(xv, [fid + 13]).astype(jnp.int32)
            ai = plsc.load_gather(xv, [fid + 14]).astype(jnp.int32)
            t = [plsc.load_gather(xv, [fid + k]) * wd[k] for k in range(13)]
            t.append(plsc.load_gather(lut, [si]))
            t.append(plsc.load_gather(lut, [ai + 32]))
            t.append(c16)
            while len(t) > 1:
                t = [t[i] + t[i + 1] for i in range(0, len(t) - 1, 2)] + (
                    [t[-1]] if len(t) % 2 else [])
            z = t[0]
            ov[pl.ds(g * _L, _L)] = _sigmoid(z)
        return carry

    lax.fori_loop(0, ngroups // _UNROLL, group, 0)
    pltpu.sync_copy(ov, out_hbm.at[pl.ds(wid * nrows, nrows)])


def kernel(x, site_table, app_table, fusion_W, fusion_b, log_W, log_b):
    B = x.shape[0]
    nw = _NC * _NS
    nrows = B // nw
    ngroups = nrows // _L
    assert nrows * nw == B and ngroups * _L == nrows and ngroups % 4 == 0

    wblk = jnp.concatenate(
        [fusion_W.astype(jnp.float32).reshape(-1),
         fusion_b.astype(jnp.float32),
         log_W.astype(jnp.float32).reshape(-1),
         log_b.astype(jnp.float32),
         jnp.zeros((14,), jnp.float32)])

    xflat = x.astype(jnp.float32).reshape(-1)
    sflat = site_table.astype(jnp.float32).reshape(-1)
    aflat = app_table.astype(jnp.float32).reshape(-1)

    run = pl.kernel(
        functools.partial(_sc_body, nrows, ngroups),
        out_type=jax.ShapeDtypeStruct((B,), jnp.float32),
        mesh=plsc.VectorSubcoreMesh(core_axis_name="c", subcore_axis_name="s"),
        compiler_params=pltpu.CompilerParams(needs_layout_passes=False),
        scratch_types=[
            pltpu.VMEM((nrows * 15,), jnp.float32),
            pltpu.VMEM((144,), jnp.float32),
            pltpu.VMEM((192,), jnp.float32),
            pltpu.VMEM((112,), jnp.float32),
            pltpu.VMEM((64,), jnp.float32),
            pltpu.VMEM((nrows,), jnp.float32),
            pltpu.VMEM_SHARED((144,), jnp.float32),
            pltpu.VMEM_SHARED((192,), jnp.float32),
            pltpu.VMEM_SHARED((112,), jnp.float32),
            pltpu.SemaphoreType.DMA,
            pltpu.SemaphoreType.DMA,
        ],
    )
    out = run(xflat, sflat, aflat, wblk)
    return out.reshape(B, 1)


# speedup vs baseline: 1.0341x; 1.0341x over previous
import functools

import jax
import jax.numpy as jnp
from jax import lax
from jax.experimental import pallas as pl
from jax.experimental.pallas import tpu as pltpu
from jax.experimental.pallas import tpu_sc as plsc

_NC = 2
_NS = 16
_L = 16


def _sigmoid(z):
    d = 1.0 + jnp.exp(jnp.minimum(-z, 87.0))
    rc = lax.bitcast_convert_type(
        jnp.int32(0x7EF311C3) - lax.bitcast_convert_type(d, jnp.int32),
        jnp.float32)
    for _ in range(3):
        rc = rc * (2.0 - d * rc)
    return rc


def _sc_body(nrows, ngroups, x_hbm, consts_hbm, out_hbm,
             xv, cv, lut, ov, spm, sem, sem2):
    wid = lax.axis_index("s") * _NC + lax.axis_index("c")
    xcp = pltpu.async_copy(
        x_hbm.at[pl.ds(wid * (nrows * 15), nrows * 15)], xv, sem)

    @pl.when(lax.axis_index("s") == 0)
    def _():
        pltpu.sync_copy(consts_hbm, spm)
    plsc.subcore_barrier()
    pltpu.sync_copy(spm, cv)

    def splat(val):
        return jnp.broadcast_to(val, (_L,))

    _W0 = 336
    wvec = [cv[pl.ds(_W0 + 16 * q, _L)] for q in range(7)]

    def wsc(flat):
        return wvec[flat // _L][flat % _L]

    w6 = [wsc(78 + j) for j in range(6)]

    vsp = []
    for d in range(12):
        acc = w6[0] * wsc(d)
        for j in range(1, 6):
            acc = acc + w6[j] * wsc(j * 12 + d)
        vsp.append(splat(acc))

    lane = jax.lax.iota(jnp.int32, _L)
    for half in range(2):
        row6 = (lane + half * _L) * 6
        ss = jnp.zeros((_L,), jnp.float32)
        aa = jnp.zeros((_L,), jnp.float32)
        for d in range(6):
            if half == 0:
                ss = ss + vsp[d] * plsc.load_gather(cv, [row6 + d])
            else:
                ss = ss + vsp[d] * plsc.load_gather(
                    cv, [jnp.minimum(row6 + d, 143)])
            aa = aa + vsp[6 + d] * plsc.load_gather(cv, [144 + row6 + d])
        lut[pl.ds(half * _L, _L)] = ss
        lut[pl.ds(32 + half * _L, _L)] = aa

    plsc.subcore_barrier()

    cs = wsc(97)
    for i in range(6):
        cs = cs + w6[i] * wsc(72 + i)
    c16 = splat(cs)
    wd = [splat(wsc(84 + k)) for k in range(13)]

    lane15 = lane * 15
    xcp.wait()

    _UNROLL = 4

    def group(gq, carry):
        for u in range(_UNROLL):
            g = gq * _UNROLL + u
            fid = lane15 + g * (_L * 15)
            si = plsc.load_gather(xv, [fid + 13]).astype(jnp.int32)
            ai = plsc.load_gather(xv, [fid + 14]).astype(jnp.int32)
            t = [plsc.load_gather(xv, [fid + k]) * wd[k] for k in range(13)]
            t.append(plsc.load_gather(lut, [si]))
            t.append(plsc.load_gather(lut, [ai + 32]))
            t.append(c16)
            while len(t) > 1:
                t = [t[i] + t[i + 1] for i in range(0, len(t) - 1, 2)] + (
                    [t[-1]] if len(t) % 2 else [])
            z = t[0]
            ov[pl.ds(g * _L, _L)] = _sigmoid(z)
        return carry

    lax.fori_loop(0, ngroups // _UNROLL, group, 0)
    pltpu.sync_copy(ov, out_hbm.at[pl.ds(wid * nrows, nrows)])


def kernel(x, site_table, app_table, fusion_W, fusion_b, log_W, log_b):
    B = x.shape[0]
    nw = _NC * _NS
    nrows = B // nw
    ngroups = nrows // _L
    assert nrows * nw == B and ngroups * _L == nrows and ngroups % 4 == 0

    consts = jnp.concatenate(
        [site_table.astype(jnp.float32).reshape(-1),
         app_table.astype(jnp.float32).reshape(-1),
         fusion_W.astype(jnp.float32).reshape(-1),
         fusion_b.astype(jnp.float32),
         log_W.astype(jnp.float32).reshape(-1),
         log_b.astype(jnp.float32),
         jnp.zeros((14,), jnp.float32)])

    xflat = x.astype(jnp.float32).reshape(-1)

    run = pl.kernel(
        functools.partial(_sc_body, nrows, ngroups),
        out_type=jax.ShapeDtypeStruct((B,), jnp.float32),
        mesh=plsc.VectorSubcoreMesh(core_axis_name="c", subcore_axis_name="s"),
        compiler_params=pltpu.CompilerParams(needs_layout_passes=False),
        scratch_types=[
            pltpu.VMEM((nrows * 15,), jnp.float32),
            pltpu.VMEM((448,), jnp.float32),
            pltpu.VMEM((64,), jnp.float32),
            pltpu.VMEM((nrows,), jnp.float32),
            pltpu.VMEM_SHARED((448,), jnp.float32),
            pltpu.SemaphoreType.DMA,
            pltpu.SemaphoreType.DMA,
        ],
    )
    out = run(xflat, consts)
    return out.reshape(B, 1)
